# Initial kernel scaffold; baseline (speedup 1.0000x reference)
import jax
import jax.numpy as jnp
from jax.experimental import pallas as pl


def kernel(idx, table):
    raise NotImplementedError("write your pallas kernel here")


# speedup vs baseline: 1.0305x; 1.0305x over previous
import functools

import jax
import jax.numpy as jnp
from jax import lax
from jax.experimental import pallas as pl
from jax.experimental.pallas import tpu as pltpu
from jax.experimental.pallas import tpu_sc as plsc

_ROWS = 51200
_D = 1000
_CHUNK = 40
_NBUF = 3


@functools.lru_cache(maxsize=None)
def _make_gather():
    info = plsc.get_sparse_core_info()
    nw = info.num_cores * info.num_subcores
    b_per_w = _ROWS // nw
    n_chunks = b_per_w // _CHUNK
    mesh = plsc.VectorSubcoreMesh(core_axis_name="c", subcore_axis_name="s")

    @functools.partial(
        pl.kernel,
        mesh=mesh,
        out_type=jax.ShapeDtypeStruct((_ROWS, _D), jnp.float32),
        compiler_params=pltpu.CompilerParams(use_tc_tiling_on_sc=False),
        scratch_types=(
            [pltpu.VMEM((b_per_w,), jnp.int32)]
            + [pltpu.VMEM((_CHUNK, _D), jnp.float32)] * _NBUF
            + [pltpu.SemaphoreType.DMA] * (2 * _NBUF)
        ),
    )
    def gather_kernel(idx_hbm, table_hbm, out_hbm, idx_v, *bufs_and_sems):
        rbufs = bufs_and_sems[:_NBUF]
        gsems = bufs_and_sems[_NBUF:2 * _NBUF]
        osems = bufs_and_sems[2 * _NBUF:]
        wid = lax.axis_index("s") * info.num_cores + lax.axis_index("c")
        base = wid * b_per_w
        pltpu.sync_copy(idx_hbm.at[pl.ds(base, b_per_w)], idx_v)

        def start_gather(c):
            b = c % _NBUF
            return pltpu.async_copy(
                table_hbm.at[idx_v.at[pl.ds(c * _CHUNK, _CHUNK)]],
                rbufs[b], gsems[b])

        gh = [None] * n_chunks
        oh = [None] * n_chunks
        for c in range(min(_NBUF, n_chunks)):
            gh[c] = start_gather(c)
        for c in range(n_chunks):
            b = c % _NBUF
            gh[c].wait()
            oh[c] = pltpu.async_copy(
                rbufs[b],
                out_hbm.at[pl.ds(base + c * _CHUNK, _CHUNK)],
                osems[b])
            if c + _NBUF < n_chunks:
                oh[c].wait()
                gh[c + _NBUF] = start_gather(c + _NBUF)
        for c in range(max(0, n_chunks - _NBUF), n_chunks):
            oh[c].wait()

    return gather_kernel


def kernel(idx, table):
    b, s = idx.shape
    flat_idx = idx.reshape(-1).astype(jnp.int32)
    out = _make_gather()(flat_idx, table)
    return out.reshape(b, s, _D)


# speedup vs baseline: 1.4148x; 1.3729x over previous
import functools

import jax
import jax.numpy as jnp
from jax import lax
from jax.experimental import pallas as pl
from jax.experimental.pallas import tpu as pltpu
from jax.experimental.pallas import tpu_sc as plsc

_ROWS = 51200
_D = 1000
_DPAD = 1024
_CHUNK = 40
_NBUF = 3


@functools.lru_cache(maxsize=None)
def _make_gather():
    info = plsc.get_sparse_core_info()
    nw = info.num_cores * info.num_subcores
    b_per_w = _ROWS // nw
    n_chunks = b_per_w // _CHUNK
    mesh = plsc.VectorSubcoreMesh(core_axis_name="c", subcore_axis_name="s")

    @functools.partial(
        pl.kernel,
        mesh=mesh,
        out_type=jax.ShapeDtypeStruct((_ROWS, _DPAD), jnp.float32),
        scratch_types=(
            [pltpu.VMEM((b_per_w,), jnp.int32)]
            + [pltpu.VMEM((_CHUNK, _DPAD), jnp.float32)] * _NBUF
            + [pltpu.SemaphoreType.DMA] * (2 * _NBUF)
        ),
    )
    def gather_kernel(idx_hbm, table_hbm, out_hbm, idx_v, *bufs_and_sems):
        rbufs = bufs_and_sems[:_NBUF]
        gsems = bufs_and_sems[_NBUF:2 * _NBUF]
        osems = bufs_and_sems[2 * _NBUF:]
        wid = lax.axis_index("s") * info.num_cores + lax.axis_index("c")
        base = wid * b_per_w
        pltpu.sync_copy(idx_hbm.at[pl.ds(base, b_per_w)], idx_v)

        def start_gather(c):
            b = c % _NBUF
            return pltpu.async_copy(
                table_hbm.at[idx_v.at[pl.ds(c * _CHUNK, _CHUNK)]],
                rbufs[b], gsems[b])

        gh = [None] * n_chunks
        oh = [None] * n_chunks
        for c in range(min(_NBUF, n_chunks)):
            gh[c] = start_gather(c)
        for c in range(n_chunks):
            b = c % _NBUF
            gh[c].wait()
            oh[c] = pltpu.async_copy(
                rbufs[b],
                out_hbm.at[pl.ds(base + c * _CHUNK, _CHUNK)],
                osems[b])
            if c + _NBUF < n_chunks:
                oh[c].wait()
                gh[c + _NBUF] = start_gather(c + _NBUF)
        for c in range(max(0, n_chunks - _NBUF), n_chunks):
            oh[c].wait()

    return gather_kernel


def kernel(idx, table):
    b, s = idx.shape
    flat_idx = idx.reshape(-1).astype(jnp.int32)
    table_pad = jnp.pad(table, ((0, 0), (0, _DPAD - _D)))
    out = _make_gather()(flat_idx, table_pad)
    return out[:, :_D].reshape(b, s, _D)
